# Initial kernel scaffold; baseline (speedup 1.0000x reference)
import jax
import jax.numpy as jnp
from jax.experimental import pallas as pl


def kernel(hd, mask):
    raise NotImplementedError("write your pallas kernel here")


# speedup vs baseline: 1.8109x; 1.8109x over previous
import functools

import jax
import jax.numpy as jnp
from jax import lax
from jax.experimental import pallas as pl
from jax.experimental.pallas import tpu as pltpu
from jax.experimental.pallas import tpu_sc as plsc

B = 16
L = 1024
D = 4096
T = B * L
NC, NS = 2, 16
NW = NC * NS
RPW = T // NW
WPB = L // RPW
CH = 8
NBUF = 3
NCHUNK = RPW // CH


def _sc_body(hd_hbm, mask_hbm, out_hbm, mask_v, idx_v, bufs,
             sem_in, sem_out):
    cid = lax.axis_index("c")
    sid = lax.axis_index("s")
    wid = sid * NC + cid
    i = wid // WPB
    j0 = (wid % WPB) * RPW

    pltpu.sync_copy(mask_hbm, mask_v)
    k_lo = i * (L // 16)
    k_hi = k_lo + L // 16

    def sum_range(lo, hi):
        def body(k, acc):
            return acc + mask_v[pl.ds(k * 16, 16)]
        acc = lax.fori_loop(lo, hi, body, jnp.zeros((16,), jnp.int32))
        s = acc[0]
        for lane in range(1, 16):
            s = s + acc[lane]
        return s

    pre = sum_range(0, k_lo)
    slen = sum_range(k_lo, k_hi)

    vcnt = jnp.clip(slen - j0, 0, RPW)
    src0 = pre + j0
    dst0 = wid * RPW

    iota = lax.iota(jnp.int32, 16)

    @pl.loop(0, RPW // 16)
    def _(q):
        r = q * 16 + iota
        idx_v[pl.ds(q * 16, 16)] = jnp.where(r < vcnt, src0 + r, T - 1)

    def out_copy(c):
        b = lax.rem(c, NBUF)
        return pltpu.make_async_copy(
            bufs.at[b], out_hbm.at[pl.ds(dst0 + c * CH, CH)], sem_out)

    def run_ring(in_copy):
        in_copy(0).start()

        @pl.loop(0, NCHUNK)
        def _(c):
            @pl.when(c + 1 < NCHUNK)
            def _():
                @pl.when(c + 1 >= NBUF)
                def _():
                    out_copy(c + 1 - NBUF).wait()
                in_copy(c + 1).start()
            in_copy(c).wait()
            out_copy(c).start()

        @pl.loop(NCHUNK - NBUF, NCHUNK)
        def _(q):
            out_copy(q).wait()

    def in_gather(c):
        b = lax.rem(c, NBUF)
        return pltpu.make_async_copy(
            hd_hbm.at[idx_v.at[pl.ds(c * CH, CH)]], bufs.at[b], sem_in)

    run_ring(in_gather)

    @pl.loop(vcnt // CH, NCHUNK)
    def _(c):
        nv = jnp.clip(vcnt - c * CH, 0, CH)
        pltpu.make_async_copy(
            hd_hbm.at[idx_v.at[pl.ds(c * CH, CH)]], bufs.at[0], sem_in).wait()
        for r in range(CH):
            @pl.when(r >= nv)
            def _():
                @pl.loop(0, D // 16)
                def _(q):
                    bufs[0, r, pl.ds(q * 16, 16)] = jnp.zeros((16,), jnp.float32)
        pltpu.make_async_copy(
            bufs.at[0], out_hbm.at[pl.ds(dst0 + c * CH, CH)], sem_out).wait()


@functools.partial(
    pl.kernel,
    out_type=jax.ShapeDtypeStruct((T, D), jnp.float32),
    mesh=plsc.VectorSubcoreMesh(
        core_axis_name="c", subcore_axis_name="s",
        num_cores=NC, num_subcores=NS),
    scratch_types=[
        pltpu.VMEM((T,), jnp.int32),
        pltpu.VMEM((RPW,), jnp.int32),
        pltpu.VMEM((NBUF, CH, D), jnp.float32),
        pltpu.SemaphoreType.DMA,
        pltpu.SemaphoreType.DMA,
    ],
)
def _sc_unpack(hd_hbm, mask_hbm, out_hbm, mask_v, idx_v, bufs,
               sem_in, sem_out):
    _sc_body(hd_hbm, mask_hbm, out_hbm, mask_v, idx_v, bufs,
             sem_in, sem_out)


def kernel(hd, mask):
    hd_flat = hd.reshape(T, D)
    mask32 = mask.astype(jnp.int32).reshape(-1)
    out = _sc_unpack(hd_flat, mask32)
    return out.reshape(B, L, D)
